# baseline (speedup 1.0000x reference)
import functools

import jax
import jax.numpy as jnp
from jax import lax
from jax.experimental import pallas as pl
from jax.experimental.pallas import tpu as pltpu
from jax.experimental.pallas import tpu_sc as plsc

_F32 = jnp.float32

_N = 10000
_E = 320000
_CH = 128

_W = 128
_NCHUNK = _E // _W
_NWORK = 32
_ITERS = -(-_NCHUNK // _NWORK)
_NPAD = 10240
_RPS = _NPAD // 16

_BE = 2000
_NBLK = _E // _BE

_vector_mesh = plsc.VectorSubcoreMesh(
    core_axis_name="core", subcore_axis_name="subcore")


def _gather_body(x_hbm, ridx_hbm, cidx_hbm, src_hbm, dst_hbm, s1, s2):
    def body(ri_v, ci_v, src_v, dst_v):
        a = pltpu.async_copy(x_hbm.at[ri_v.at[0]], src_v, s1)
        b = pltpu.async_copy(x_hbm.at[ci_v.at[0]], dst_v, s2)
        a.wait()
        b.wait()

    pltpu.emit_pipeline(
        body,
        grid=(ridx_hbm.shape[1] // _W,),
        in_specs=[pl.BlockSpec((1, _W), lambda i: (0, i)),
                  pl.BlockSpec((1, _W), lambda i: (0, i))],
        out_specs=[pl.BlockSpec((_W, _CH), lambda i: (i, 0)),
                   pl.BlockSpec((_W, _CH), lambda i: (i, 0))],
        core_axis_name=("core", "subcore"),
        dimension_semantics=(pltpu.PARALLEL,),
    )(ridx_hbm, cidx_hbm, src_hbm, dst_hbm)


@jax.jit
def _sc_gather(x, ridx, cidx):
    ne = ridx.shape[1]
    f = pl.kernel(
        _gather_body,
        out_type=[jax.ShapeDtypeStruct((ne, _CH), _F32),
                  jax.ShapeDtypeStruct((ne, _CH), _F32)],
        mesh=_vector_mesh,
        scratch_types=[pltpu.SemaphoreType.DMA,
                       pltpu.SemaphoreType.DMA],
    )
    return f(x, ridx, cidx)


def _edges_body(src_ref, dst_ref, ea_ref,
                wa_ref, wb_ref, wc_ref, be1_ref, we2_ref, be2_ref,
                wna_ref, wnb_ref, bn1_ref, wn2_ref, bn2_ref,
                en_ref, no_ref):
    src = src_ref[...]
    dst = dst_ref[...]
    ea = ea_ref[...]
    dot = functools.partial(jnp.dot, preferred_element_type=_F32)
    e_h = jnp.maximum(
        dot(src, wa_ref[...]) + dot(dst, wb_ref[...]) + dot(ea, wc_ref[...])
        + be1_ref[...], 0.0)
    edge_new = dot(e_h, we2_ref[...]) + be2_ref[...]
    en_ref[...] = edge_new
    n_h = jnp.maximum(
        dot(dst, wna_ref[...]) + dot(edge_new, wnb_ref[...]) + bn1_ref[...],
        0.0)
    no_ref[...] = dot(n_h, wn2_ref[...]) + bn2_ref[...]


@jax.jit
def _tc_edges(src, dst, ea, wa, wb, wc, be1, we2, be2, wna, wnb, bn1, wn2,
              bn2):
    blk = lambda i: (i, 0)
    full = lambda i: (0, 0)
    e_spec = pl.BlockSpec((_BE, _CH), blk)
    w_spec = pl.BlockSpec((_CH, _CH), full)
    b_spec = pl.BlockSpec((1, _CH), full)
    ne = src.shape[0]
    return pl.pallas_call(
        _edges_body,
        grid=(ne // _BE,),
        in_specs=[e_spec, e_spec, e_spec,
                  w_spec, w_spec, w_spec, b_spec, w_spec, b_spec,
                  w_spec, w_spec, b_spec, w_spec, b_spec],
        out_specs=[e_spec, e_spec],
        out_shape=[jax.ShapeDtypeStruct((ne, _CH), _F32),
                   jax.ShapeDtypeStruct((ne, _CH), _F32)],
    )(src, dst, ea, wa, wb, wc, be1, we2, be2, wna, wnb, bn1, wn2, bn2)


_WG = 64


def _scatter_sums_body(nout_hbm, ridx_hbm, zsum_hbm, sums_hbm,
                       acc_shr, stg_v):
    cid = lax.axis_index("core")
    sid = lax.axis_index("subcore")
    r0 = sid * _RPS
    nstripe = _RPS // _WG

    pltpu.sync_copy(zsum_hbm, stg_v)
    for j in range(nstripe):
        pltpu.sync_copy(stg_v, acc_shr.at[pl.ds(r0 + j * _WG, _WG)])
    plsc.subcore_barrier()

    def body(ri_v, val_v):
        pltpu.sync_copy(val_v, acc_shr.at[ri_v.at[0]], add=True)

    pltpu.emit_pipeline(
        body,
        grid=(ridx_hbm.shape[1] // _W,),
        in_specs=[pl.BlockSpec((1, _W), lambda i: (0, i)),
                  pl.BlockSpec((_W, _CH), lambda i: (i, 0))],
        out_specs=[],
        core_axis_name=("core", "subcore"),
        dimension_semantics=(pltpu.PARALLEL,),
    )(ridx_hbm, nout_hbm)

    plsc.subcore_barrier()
    for j in range(nstripe):
        pltpu.sync_copy(acc_shr.at[pl.ds(r0 + j * _WG, _WG)], stg_v)
        pltpu.sync_copy(stg_v, sums_hbm.at[cid, pl.ds(r0 + j * _WG, _WG)])


@jax.jit
def _sc_scatter_sums(nout, ridx, zsum):
    f = pl.kernel(
        _scatter_sums_body,
        out_type=jax.ShapeDtypeStruct((2, _NPAD, _CH), _F32),
        mesh=_vector_mesh,
        scratch_types=[pltpu.VMEM_SHARED((_NPAD, _CH), _F32),
                       pltpu.VMEM((_WG, _CH), _F32)],
    )
    return f(nout, ridx, zsum)


def _scatter_cnts_body(ridx_hbm, zcnt_hbm, ones_hbm, cnts_hbm,
                       cnt_shr, one_v):
    cid = lax.axis_index("core")
    sid = lax.axis_index("subcore")
    r0 = sid * _RPS
    nstripe = _RPS // _W

    pltpu.sync_copy(zcnt_hbm, one_v)
    for j in range(nstripe):
        pltpu.sync_copy(one_v, cnt_shr.at[pl.ds(r0 + j * _W, _W)])
    pltpu.sync_copy(ones_hbm, one_v)
    plsc.subcore_barrier()

    def body(ri_v):
        pltpu.sync_copy(one_v, cnt_shr.at[ri_v.at[0]], add=True)

    pltpu.emit_pipeline(
        body,
        grid=(_NCHUNK,),
        in_specs=[pl.BlockSpec((1, _W), lambda i: (0, i))],
        out_specs=[],
        core_axis_name=("core", "subcore"),
        dimension_semantics=(pltpu.PARALLEL,),
    )(ridx_hbm)

    plsc.subcore_barrier()
    for j in range(nstripe):
        pltpu.sync_copy(cnt_shr.at[pl.ds(r0 + j * _W, _W)], one_v)
        pltpu.sync_copy(one_v, cnts_hbm.at[cid, pl.ds(r0 + j * _W, _W)])


@jax.jit
def _sc_scatter_cnts(ridx, zcnt, ones):
    f = pl.kernel(
        _scatter_cnts_body,
        out_type=jax.ShapeDtypeStruct((2, _NPAD, _CH), _F32),
        mesh=_vector_mesh,
        scratch_types=[pltpu.VMEM_SHARED((_NPAD, _CH), _F32),
                       pltpu.VMEM((_W, _CH), _F32)],
    )
    return f(ridx, zcnt, ones)


def _fin_body(sa_ref, sb_ref, c_ref, o_ref):
    s = (sa_ref[0, :_N, :] + sa_ref[1, :_N, :]
         + sb_ref[0, :_N, :] + sb_ref[1, :_N, :])
    c = c_ref[0, :_N, 0:1] + c_ref[1, :_N, 0:1]
    o_ref[...] = s / jnp.maximum(c, 1.0)


@jax.jit
def _tc_finalize(sums_a, sums_b, cnts):
    return pl.pallas_call(
        _fin_body,
        out_shape=jax.ShapeDtypeStruct((_N, _CH), _F32),
    )(sums_a, sums_b, cnts)


def kernel(x, edge_index, edge_attr, We1, be1, We2, be2, Wn1, bn1, Wn2, bn2):
    idx = edge_index.astype(jnp.int32)
    ridx = idx[0].reshape(1, _E)
    cidx = idx[1].reshape(1, _E)
    half = _E // 2
    ridx_a, ridx_b = ridx[:, :half], ridx[:, half:]
    cidx_a, cidx_b = cidx[:, :half], cidx[:, half:]
    ea_a, ea_b = edge_attr[:half], edge_attr[half:]

    wa, wb, wc = We1[:_CH], We1[_CH:2 * _CH], We1[2 * _CH:]
    wna, wnb = Wn1[:_CH], Wn1[_CH:]
    wargs = (wa, wb, wc, be1.reshape(1, _CH), We2, be2.reshape(1, _CH),
             wna, wnb, bn1.reshape(1, _CH), Wn2, bn2.reshape(1, _CH))

    zsum = jnp.zeros((_WG, _CH), _F32)
    zcnt = jnp.zeros((_W, _CH), _F32)
    ones = jnp.ones((_W, _CH), _F32)

    src_a, dst_a = _sc_gather(x, ridx_a, cidx_a)
    src_b, dst_b = _sc_gather(x, ridx_b, cidx_b)
    en_a, no_a = _tc_edges(src_a, dst_a, ea_a, *wargs)
    en_b, no_b = _tc_edges(src_b, dst_b, ea_b, *wargs)
    cnts = _sc_scatter_cnts(ridx, zcnt, ones)
    sums_a = _sc_scatter_sums(no_a, ridx_a, zsum)
    sums_b = _sc_scatter_sums(no_b, ridx_b, zsum)

    x_new = _tc_finalize(sums_a, sums_b, cnts)
    edge_new = jnp.concatenate([en_a, en_b], axis=0)
    return (x_new, edge_index, edge_new)


# speedup vs baseline: 1.1622x; 1.1622x over previous
import functools

import jax
import jax.numpy as jnp
from jax import lax
from jax.experimental import pallas as pl
from jax.experimental.pallas import tpu as pltpu
from jax.experimental.pallas import tpu_sc as plsc

_F32 = jnp.float32

_N = 10000
_E = 320000
_CH = 128

_W = 128
_NCHUNK = _E // _W
_NWORK = 32
_ITERS = -(-_NCHUNK // _NWORK)
_NPAD = 10240
_RPS = _NPAD // 16

_BE = 2000
_NBLK = _E // _BE

_vector_mesh = plsc.VectorSubcoreMesh(
    core_axis_name="core", subcore_axis_name="subcore")


def _gather_body(x_hbm, ridx_hbm, cidx_hbm, src_hbm, dst_hbm, s1, s2):
    def body(ri_v, ci_v, src_v, dst_v):
        a = pltpu.async_copy(x_hbm.at[ri_v.at[0]], src_v, s1)
        b = pltpu.async_copy(x_hbm.at[ci_v.at[0]], dst_v, s2)
        a.wait()
        b.wait()

    pltpu.emit_pipeline(
        body,
        grid=(ridx_hbm.shape[1] // _W,),
        in_specs=[pl.BlockSpec((1, _W), lambda i: (0, i)),
                  pl.BlockSpec((1, _W), lambda i: (0, i))],
        out_specs=[pl.BlockSpec((_W, _CH), lambda i: (i, 0)),
                   pl.BlockSpec((_W, _CH), lambda i: (i, 0))],
        core_axis_name=("core", "subcore"),
        dimension_semantics=(pltpu.PARALLEL,),
    )(ridx_hbm, cidx_hbm, src_hbm, dst_hbm)


@jax.jit
def _sc_gather(x, ridx, cidx):
    ne = ridx.shape[1]
    f = pl.kernel(
        _gather_body,
        out_type=[jax.ShapeDtypeStruct((ne, _CH), _F32),
                  jax.ShapeDtypeStruct((ne, _CH), _F32)],
        mesh=_vector_mesh,
        scratch_types=[pltpu.SemaphoreType.DMA,
                       pltpu.SemaphoreType.DMA],
    )
    return f(x, ridx, cidx)


def _edges_body(src_ref, dst_ref, ea_ref,
                wa_ref, wb_ref, wc_ref, be1_ref, we2_ref, be2_ref,
                wna_ref, wnb_ref, bn1_ref, wn2_ref, bn2_ref,
                en_ref, no_ref):
    src = src_ref[...]
    dst = dst_ref[...]
    ea = ea_ref[...]
    dot = functools.partial(jnp.dot, preferred_element_type=_F32)
    e_h = jnp.maximum(
        dot(src, wa_ref[...]) + dot(dst, wb_ref[...]) + dot(ea, wc_ref[...])
        + be1_ref[...], 0.0)
    edge_new = dot(e_h, we2_ref[...]) + be2_ref[...]
    en_ref[...] = edge_new
    n_h = jnp.maximum(
        dot(dst, wna_ref[...]) + dot(edge_new, wnb_ref[...]) + bn1_ref[...],
        0.0)
    no_ref[...] = dot(n_h, wn2_ref[...]) + bn2_ref[...]


@jax.jit
def _tc_edges(src, dst, ea, wa, wb, wc, be1, we2, be2, wna, wnb, bn1, wn2,
              bn2):
    blk = lambda i: (i, 0)
    full = lambda i: (0, 0)
    e_spec = pl.BlockSpec((_BE, _CH), blk)
    w_spec = pl.BlockSpec((_CH, _CH), full)
    b_spec = pl.BlockSpec((1, _CH), full)
    ne = src.shape[0]
    return pl.pallas_call(
        _edges_body,
        grid=(ne // _BE,),
        in_specs=[e_spec, e_spec, e_spec,
                  w_spec, w_spec, w_spec, b_spec, w_spec, b_spec,
                  w_spec, w_spec, b_spec, w_spec, b_spec],
        out_specs=[e_spec, e_spec],
        out_shape=[jax.ShapeDtypeStruct((ne, _CH), _F32),
                   jax.ShapeDtypeStruct((ne, _CH), _F32)],
    )(src, dst, ea, wa, wb, wc, be1, we2, be2, wna, wnb, bn1, wn2, bn2)


_WG = 64


def _scatter_sums_body(nout_hbm, ridx_hbm, zsum_hbm, sums_hbm,
                       acc_shr, stg_v):
    cid = lax.axis_index("core")
    sid = lax.axis_index("subcore")
    r0 = sid * _RPS
    nstripe = _RPS // _WG

    pltpu.sync_copy(zsum_hbm, stg_v)
    for j in range(nstripe):
        pltpu.sync_copy(stg_v, acc_shr.at[pl.ds(r0 + j * _WG, _WG)])
    plsc.subcore_barrier()

    def body(ri_v, val_v):
        pltpu.sync_copy(val_v, acc_shr.at[ri_v.at[0]], add=True)

    pltpu.emit_pipeline(
        body,
        grid=(ridx_hbm.shape[1] // _W,),
        in_specs=[pl.BlockSpec((1, _W), lambda i: (0, i)),
                  pl.BlockSpec((_W, _CH), lambda i: (i, 0))],
        out_specs=[],
        core_axis_name=("core", "subcore"),
        dimension_semantics=(pltpu.PARALLEL,),
    )(ridx_hbm, nout_hbm)

    plsc.subcore_barrier()
    for j in range(nstripe):
        pltpu.sync_copy(acc_shr.at[pl.ds(r0 + j * _WG, _WG)], stg_v)
        pltpu.sync_copy(stg_v, sums_hbm.at[cid, pl.ds(r0 + j * _WG, _WG)])


@jax.jit
def _sc_scatter_sums(nout, ridx, zsum):
    f = pl.kernel(
        _scatter_sums_body,
        out_type=jax.ShapeDtypeStruct((2, _NPAD, _CH), _F32),
        mesh=_vector_mesh,
        scratch_types=[pltpu.VMEM_SHARED((_NPAD, _CH), _F32),
                       pltpu.VMEM((_WG, _CH), _F32)],
    )
    return f(nout, ridx, zsum)


def _scatter_cnts_body(ridx_hbm, zcnt_hbm, ones_hbm, cnts_hbm,
                       cnt_shr, one_v):
    cid = lax.axis_index("core")
    sid = lax.axis_index("subcore")
    r0 = sid * _RPS
    nstripe = _RPS // _W

    pltpu.sync_copy(zcnt_hbm, one_v)
    for j in range(nstripe):
        pltpu.sync_copy(one_v, cnt_shr.at[pl.ds(r0 + j * _W, _W)])
    pltpu.sync_copy(ones_hbm, one_v)
    plsc.subcore_barrier()

    def body(ri_v):
        pltpu.sync_copy(one_v, cnt_shr.at[ri_v.at[0]], add=True)

    pltpu.emit_pipeline(
        body,
        grid=(_NCHUNK,),
        in_specs=[pl.BlockSpec((1, _W), lambda i: (0, i))],
        out_specs=[],
        core_axis_name=("core", "subcore"),
        dimension_semantics=(pltpu.PARALLEL,),
    )(ridx_hbm)

    plsc.subcore_barrier()
    for j in range(nstripe):
        pltpu.sync_copy(cnt_shr.at[pl.ds(r0 + j * _W, _W)], one_v)
        pltpu.sync_copy(one_v, cnts_hbm.at[cid, pl.ds(r0 + j * _W, _W)])


@jax.jit
def _sc_scatter_cnts(ridx, zcnt, ones):
    f = pl.kernel(
        _scatter_cnts_body,
        out_type=jax.ShapeDtypeStruct((2, _NPAD, _CH), _F32),
        mesh=_vector_mesh,
        scratch_types=[pltpu.VMEM_SHARED((_NPAD, _CH), _F32),
                       pltpu.VMEM((_W, _CH), _F32)],
    )
    return f(ridx, zcnt, ones)


def _fin_body(s_ref, c_ref, o_ref):
    s = s_ref[0, :_N, :] + s_ref[1, :_N, :]
    c = c_ref[0, :_N, 0:1] + c_ref[1, :_N, 0:1]
    o_ref[...] = s / jnp.maximum(c, 1.0)


@jax.jit
def _tc_finalize(sums, cnts):
    return pl.pallas_call(
        _fin_body,
        out_shape=jax.ShapeDtypeStruct((_N, _CH), _F32),
    )(sums, cnts)


def kernel(x, edge_index, edge_attr, We1, be1, We2, be2, Wn1, bn1, Wn2, bn2):
    idx = edge_index.astype(jnp.int32)
    ridx = idx[0].reshape(1, _E)
    cidx = idx[1].reshape(1, _E)

    wa, wb, wc = We1[:_CH], We1[_CH:2 * _CH], We1[2 * _CH:]
    wna, wnb = Wn1[:_CH], Wn1[_CH:]
    wargs = (wa, wb, wc, be1.reshape(1, _CH), We2, be2.reshape(1, _CH),
             wna, wnb, bn1.reshape(1, _CH), Wn2, bn2.reshape(1, _CH))

    zsum = jnp.zeros((_WG, _CH), _F32)
    zcnt = jnp.zeros((_W, _CH), _F32)
    ones = jnp.ones((_W, _CH), _F32)

    src, dst = _sc_gather(x, ridx, cidx)
    edge_new, n_out = _tc_edges(src, dst, edge_attr, *wargs)
    cnts = _sc_scatter_cnts(ridx, zcnt, ones)
    sums = _sc_scatter_sums(n_out, ridx, zsum)

    x_new = _tc_finalize(sums, cnts)
    return (x_new, edge_index, edge_new)
